# baseline (speedup 1.0000x reference)
import functools

import jax
import jax.numpy as jnp
from jax import lax
from jax.experimental import pallas as pl
from jax.experimental.pallas import tpu as pltpu
from jax.experimental.pallas import tpu_sc as plsc

_D = 768
_L = 16
_NSL = _D // _L
_B = 64
_S = 512
_NC = 2
_NS = 16
_NW = _NC * _NS
_POS_W = _S // _NW
_TOK_W = _B * _POS_W
_CHUNK = _POS_W
_NCHUNK = _B
_NBUF = 4
_EPS = 1e-12


def _rsqrt16(v):
    iv = plsc.bitcast(v, jnp.int32)
    ih = jnp.int32(0x5F3759DF) - lax.shift_right_logical(iv, 1)
    y = plsc.bitcast(ih, jnp.float32)
    for _ in range(3):
        y = y * (1.5 - 0.5 * v * y * y)
    return y


def _body(word_hbm, idx_hbm, tt_hbm, pos_hbm, type_hbm, gam_hbm, bet_hbm,
          out_hbm,
          idx_v, tt_v, base2_v, type_v, gam_v, bet_v, stats_v,
          rows0, rows1, rows2, rows3,
          gsem0, gsem1, gsem2, gsem3, wsem0, wsem1, wsem2, wsem3):
    wid = lax.axis_index("s") * _NC + lax.axis_index("c")
    bufs = (rows0, rows1, rows2, rows3)
    gsems = (gsem0, gsem1, gsem2, gsem3)
    wsems = (wsem0, wsem1, wsem2, wsem3)

    pltpu.sync_copy(idx_hbm.at[pl.ds(wid * _NCHUNK, _NCHUNK), :], idx_v)
    pltpu.sync_copy(tt_hbm.at[pl.ds(wid * _TOK_W, _TOK_W)],
                    tt_v.at[pl.ds(0, _TOK_W)])
    pltpu.sync_copy(pos_hbm.at[pl.ds(wid * _POS_W, _POS_W), :],
                    base2_v.at[0])
    pltpu.sync_copy(pos_hbm.at[pl.ds(wid * _POS_W, _POS_W), :],
                    base2_v.at[1])
    pltpu.sync_copy(type_hbm, type_v)
    pltpu.sync_copy(gam_hbm, gam_v)
    pltpu.sync_copy(bet_hbm, bet_v)

    def _init_d(d, _):
        sl = pl.ds(d * _L, _L)
        t0 = type_v[0, sl]
        t1 = type_v[1, sl]

        def _init_j(j, __):
            base2_v[0, j, sl] = base2_v[0, j, sl] + t0
            base2_v[1, j, sl] = base2_v[1, j, sl] + t1
            return 0

        lax.fori_loop(0, _POS_W, _init_j, 0)
        return 0

    lax.fori_loop(0, _NSL, _init_d, 0)

    zeros = jnp.zeros((_L,), jnp.float32)
    lanes = lax.iota(jnp.int32, _L)

    def _out_at(c):
        return out_hbm.at[c, pl.ds(wid * _POS_W, _POS_W), :]

    def _compute_chunk(rows, c):
        def _p1(iq, _):
            toks = [iq * 4 + t for t in range(4)]
            ttis = [tt_v[pl.ds(c * _CHUNK + i, _L)][0] for i in toks]
            s1 = [zeros] * 4
            s2 = [zeros] * 4
            sl0 = pl.ds(0, _L)
            cur_r = [rows[i, sl0] for i in toks]
            cur_b = [base2_v[tti, i, sl0] for tti, i in zip(ttis, toks)]
            for d in range(_NSL):
                if d + 1 < _NSL:
                    sln = pl.ds((d + 1) * _L, _L)
                    nxt_r = [rows[i, sln] for i in toks]
                    nxt_b = [base2_v[tti, i, sln]
                             for tti, i in zip(ttis, toks)]
                sl = pl.ds(d * _L, _L)
                xs = [r + b for r, b in zip(cur_r, cur_b)]
                for t in range(4):
                    rows[toks[t], sl] = xs[t]
                s1 = [a + x for a, x in zip(s1, xs)]
                s2 = [a + x * x for a, x in zip(s2, xs)]
                if d + 1 < _NSL:
                    cur_r, cur_b = nxt_r, nxt_b
            for t in range(4):
                stats_v[toks[t], :] = s1[t]
                stats_v[_CHUNK + toks[t], :] = s2[t]
            return 0

        lax.fori_loop(0, _CHUNK // 4, _p1, 0)

        tot1 = zeros
        tot2 = zeros
        for l in range(_L):
            cl = jnp.full((_L,), l, jnp.int32)
            tot1 = tot1 + plsc.load_gather(stats_v, [lanes, cl])
            tot2 = tot2 + plsc.load_gather(stats_v, [lanes + _CHUNK, cl])
        mean_v = tot1 * (1.0 / _D)
        var_v = jnp.maximum(tot2 * (1.0 / _D) - mean_v * mean_v, 0.0) + _EPS
        rst_v = _rsqrt16(var_v)
        rst_s = [rst_v[j] for j in range(_L)]
        mean_s = [mean_v[j] for j in range(_L)]

        def _p2(d, _):
            sl = pl.ds(d * _L, _L)
            gsl = gam_v[sl]
            bsl = bet_v[sl]
            x = rows[0, sl]
            for j in range(_L):
                if j + 1 < _L:
                    xn = rows[j + 1, sl]
                z = (x - mean_s[j]) * rst_s[j]
                rows[j, sl] = z * gsl + bsl
                if j + 1 < _L:
                    x = xn
            return 0

        lax.fori_loop(0, _NSL, _p2, 0, unroll=2)

    pltpu.async_copy(word_hbm.at[idx_v.at[0]], bufs[0], gsems[0])
    pltpu.async_copy(word_hbm.at[idx_v.at[1]], bufs[1], gsems[1])

    def _iter(cc, _):
        for k in range(_NBUF):
            c = cc * _NBUF + k
            k2 = (k + 2) % _NBUF
            pltpu.make_async_copy(word_hbm.at[idx_v.at[c]],
                                  bufs[k], gsems[k]).wait()
            _compute_chunk(bufs[k], c)
            pltpu.async_copy(bufs[k], _out_at(c), wsems[k])

            def _drain_w(c=c, k2=k2):
                pltpu.make_async_copy(bufs[k2], _out_at(c - 2),
                                      wsems[k2]).wait()

            def _start_g(c=c, k2=k2):
                pltpu.async_copy(word_hbm.at[idx_v.at[c + 2]],
                                 bufs[k2], gsems[k2])

            if k < 2:
                pl.when(cc > 0)(_drain_w)
                _start_g()
            else:
                _drain_w()
                pl.when(cc < _NCHUNK // _NBUF - 1)(_start_g)
        return 0

    lax.fori_loop(0, _NCHUNK // _NBUF, _iter, 0)

    pltpu.make_async_copy(bufs[2], _out_at(_NCHUNK - 2), wsems[2]).wait()
    pltpu.make_async_copy(bufs[3], _out_at(_NCHUNK - 1), wsems[3]).wait()


_emb_ln = functools.partial(
    pl.kernel,
    out_type=jax.ShapeDtypeStruct((_B, _S, _D), jnp.float32),
    mesh=plsc.VectorSubcoreMesh(core_axis_name="c", subcore_axis_name="s"),
    compiler_params=pltpu.CompilerParams(needs_layout_passes=False),
    scratch_types=[
        pltpu.VMEM((_NCHUNK, _CHUNK), jnp.int32),
        pltpu.VMEM((_TOK_W + _L,), jnp.int32),
        pltpu.VMEM((2, _POS_W, _D), jnp.float32),
        pltpu.VMEM((2, _D), jnp.float32),
        pltpu.VMEM((_D,), jnp.float32),
        pltpu.VMEM((_D,), jnp.float32),
        pltpu.VMEM((2 * _CHUNK, _L), jnp.float32),
        pltpu.VMEM((_CHUNK, _D), jnp.float32),
        pltpu.VMEM((_CHUNK, _D), jnp.float32),
        pltpu.VMEM((_CHUNK, _D), jnp.float32),
        pltpu.VMEM((_CHUNK, _D), jnp.float32),
        pltpu.SemaphoreType.DMA,
        pltpu.SemaphoreType.DMA,
        pltpu.SemaphoreType.DMA,
        pltpu.SemaphoreType.DMA,
        pltpu.SemaphoreType.DMA,
        pltpu.SemaphoreType.DMA,
        pltpu.SemaphoreType.DMA,
        pltpu.SemaphoreType.DMA,
    ],
)(_body)


def kernel(input_ids, token_type_ids, word_emb, pos_emb, type_emb, gamma, beta):
    ids_p = (input_ids.astype(jnp.int32)
             .reshape(_B, _NW, _POS_W)
             .transpose(1, 0, 2)
             .reshape(_NW * _NCHUNK, _CHUNK))
    tt_p = (token_type_ids.astype(jnp.int32)
            .reshape(_B, _NW, _POS_W)
            .transpose(1, 0, 2)
            .reshape(_NW * _TOK_W))
    return _emb_ln(word_emb, ids_p, tt_p, pos_emb, type_emb, gamma, beta)


# speedup vs baseline: 1.0038x; 1.0038x over previous
import functools

import jax
import jax.numpy as jnp
from jax import lax
from jax.experimental import pallas as pl
from jax.experimental.pallas import tpu as pltpu
from jax.experimental.pallas import tpu_sc as plsc

_D = 768
_L = 16
_NSL = _D // _L
_B = 64
_S = 512
_NC = 2
_NS = 16
_NW = _NC * _NS
_POS_W = _S // _NW
_TOK_W = _B * _POS_W
_CHUNK = _POS_W
_NCHUNK = _B
_NBUF = 4
_EPS = 1e-12


def _rsqrt16(v):
    iv = plsc.bitcast(v, jnp.int32)
    ih = jnp.int32(0x5F3759DF) - lax.shift_right_logical(iv, 1)
    y = plsc.bitcast(ih, jnp.float32)
    for _ in range(3):
        y = y * (1.5 - 0.5 * v * y * y)
    return y


def _body(word_hbm, idx_hbm, tt_hbm, pos_hbm, type_hbm, gam_hbm, bet_hbm,
          out_hbm,
          idx_v, tt_v, base2_v, type_v, gam_v, bet_v, stats_v,
          rows0, rows1, rows2, rows3,
          gsem0, gsem1, gsem2, gsem3, wsem0, wsem1, wsem2, wsem3):
    wid = lax.axis_index("s") * _NC + lax.axis_index("c")
    bufs = (rows0, rows1, rows2, rows3)
    gsems = (gsem0, gsem1, gsem2, gsem3)
    wsems = (wsem0, wsem1, wsem2, wsem3)

    pltpu.sync_copy(idx_hbm.at[pl.ds(wid * _NCHUNK, _NCHUNK), :], idx_v)
    pltpu.sync_copy(tt_hbm.at[pl.ds(wid * _TOK_W, _TOK_W)],
                    tt_v.at[pl.ds(0, _TOK_W)])
    pltpu.sync_copy(pos_hbm.at[pl.ds(wid * _POS_W, _POS_W), :],
                    base2_v.at[0])
    pltpu.sync_copy(pos_hbm.at[pl.ds(wid * _POS_W, _POS_W), :],
                    base2_v.at[1])
    pltpu.sync_copy(type_hbm, type_v)
    pltpu.sync_copy(gam_hbm, gam_v)
    pltpu.sync_copy(bet_hbm, bet_v)

    def _init_d(d, _):
        sl = pl.ds(d * _L, _L)
        t0 = type_v[0, sl]
        t1 = type_v[1, sl]

        def _init_j(j, __):
            base2_v[0, j, sl] = base2_v[0, j, sl] + t0
            base2_v[1, j, sl] = base2_v[1, j, sl] + t1
            return 0

        lax.fori_loop(0, _POS_W, _init_j, 0)
        return 0

    lax.fori_loop(0, _NSL, _init_d, 0)

    zeros = jnp.zeros((_L,), jnp.float32)
    lanes = lax.iota(jnp.int32, _L)

    def _out_at(c):
        return out_hbm.at[c, pl.ds(wid * _POS_W, _POS_W), :]

    def _compute_chunk(rows, c):
        def _p1(ip, _):
            i0 = ip * 2
            i1 = i0 + 1
            tti0 = tt_v[pl.ds(c * _CHUNK + i0, _L)][0]
            tti1 = tt_v[pl.ds(c * _CHUNK + i1, _L)][0]
            s1_0 = s2_0 = s1_1 = s2_1 = zeros
            sl = pl.ds(0, _L)
            r0 = rows[i0, sl]
            b0 = base2_v[tti0, i0, sl]
            r1 = rows[i1, sl]
            b1 = base2_v[tti1, i1, sl]
            for d in range(_NSL):
                if d + 1 < _NSL:
                    sln = pl.ds((d + 1) * _L, _L)
                    rn0 = rows[i0, sln]
                    bn0 = base2_v[tti0, i0, sln]
                    rn1 = rows[i1, sln]
                    bn1 = base2_v[tti1, i1, sln]
                sl = pl.ds(d * _L, _L)
                x0 = r0 + b0
                x1 = r1 + b1
                rows[i0, sl] = x0
                rows[i1, sl] = x1
                s1_0 = s1_0 + x0
                s2_0 = s2_0 + x0 * x0
                s1_1 = s1_1 + x1
                s2_1 = s2_1 + x1 * x1
                if d + 1 < _NSL:
                    r0, b0, r1, b1 = rn0, bn0, rn1, bn1
            stats_v[i0, :] = s1_0
            stats_v[i1, :] = s1_1
            stats_v[_CHUNK + i0, :] = s2_0
            stats_v[_CHUNK + i1, :] = s2_1
            return 0

        lax.fori_loop(0, _CHUNK // 2, _p1, 0)

        tot1 = zeros
        tot2 = zeros
        for l in range(_L):
            cl = jnp.full((_L,), l, jnp.int32)
            tot1 = tot1 + plsc.load_gather(stats_v, [lanes, cl])
            tot2 = tot2 + plsc.load_gather(stats_v, [lanes + _CHUNK, cl])
        mean_v = tot1 * (1.0 / _D)
        var_v = jnp.maximum(tot2 * (1.0 / _D) - mean_v * mean_v, 0.0) + _EPS
        rst_v = _rsqrt16(var_v)
        rst_s = [rst_v[j] for j in range(_L)]
        mean_s = [mean_v[j] for j in range(_L)]

        def _p2(d, _):
            sl = pl.ds(d * _L, _L)
            gsl = gam_v[sl]
            bsl = bet_v[sl]
            x = rows[0, sl]
            for j in range(_L):
                if j + 1 < _L:
                    xn = rows[j + 1, sl]
                z = (x - mean_s[j]) * rst_s[j]
                rows[j, sl] = z * gsl + bsl
                if j + 1 < _L:
                    x = xn
            return 0

        lax.fori_loop(0, _NSL, _p2, 0, unroll=2)

    pltpu.async_copy(word_hbm.at[idx_v.at[0]], bufs[0], gsems[0])
    pltpu.async_copy(word_hbm.at[idx_v.at[1]], bufs[1], gsems[1])

    def _iter(cc, _):
        for k in range(_NBUF):
            c = cc * _NBUF + k
            k2 = (k + 2) % _NBUF
            pltpu.make_async_copy(word_hbm.at[idx_v.at[c]],
                                  bufs[k], gsems[k]).wait()
            _compute_chunk(bufs[k], c)
            pltpu.async_copy(bufs[k], _out_at(c), wsems[k])

            def _drain_w(c=c, k2=k2):
                pltpu.make_async_copy(bufs[k2], _out_at(c - 2),
                                      wsems[k2]).wait()

            def _start_g(c=c, k2=k2):
                pltpu.async_copy(word_hbm.at[idx_v.at[c + 2]],
                                 bufs[k2], gsems[k2])

            if k < 2:
                pl.when(cc > 0)(_drain_w)
                _start_g()
            else:
                _drain_w()
                pl.when(cc < _NCHUNK // _NBUF - 1)(_start_g)
        return 0

    lax.fori_loop(0, _NCHUNK // _NBUF, _iter, 0)

    pltpu.make_async_copy(bufs[2], _out_at(_NCHUNK - 2), wsems[2]).wait()
    pltpu.make_async_copy(bufs[3], _out_at(_NCHUNK - 1), wsems[3]).wait()


_emb_ln = functools.partial(
    pl.kernel,
    out_type=jax.ShapeDtypeStruct((_B, _S, _D), jnp.float32),
    mesh=plsc.VectorSubcoreMesh(core_axis_name="c", subcore_axis_name="s"),
    compiler_params=pltpu.CompilerParams(needs_layout_passes=False),
    scratch_types=[
        pltpu.VMEM((_NCHUNK, _CHUNK), jnp.int32),
        pltpu.VMEM((_TOK_W + _L,), jnp.int32),
        pltpu.VMEM((2, _POS_W, _D), jnp.float32),
        pltpu.VMEM((2, _D), jnp.float32),
        pltpu.VMEM((_D,), jnp.float32),
        pltpu.VMEM((_D,), jnp.float32),
        pltpu.VMEM((2 * _CHUNK, _L), jnp.float32),
        pltpu.VMEM((_CHUNK, _D), jnp.float32),
        pltpu.VMEM((_CHUNK, _D), jnp.float32),
        pltpu.VMEM((_CHUNK, _D), jnp.float32),
        pltpu.VMEM((_CHUNK, _D), jnp.float32),
        pltpu.SemaphoreType.DMA,
        pltpu.SemaphoreType.DMA,
        pltpu.SemaphoreType.DMA,
        pltpu.SemaphoreType.DMA,
        pltpu.SemaphoreType.DMA,
        pltpu.SemaphoreType.DMA,
        pltpu.SemaphoreType.DMA,
        pltpu.SemaphoreType.DMA,
    ],
)(_body)


def kernel(input_ids, token_type_ids, word_emb, pos_emb, type_emb, gamma, beta):
    ids_p = (input_ids.astype(jnp.int32)
             .reshape(_B, _NW, _POS_W)
             .transpose(1, 0, 2)
             .reshape(_NW * _NCHUNK, _CHUNK))
    tt_p = (token_type_ids.astype(jnp.int32)
            .reshape(_B, _NW, _POS_W)
            .transpose(1, 0, 2)
            .reshape(_NW * _TOK_W))
    return _emb_ln(word_emb, ids_p, tt_p, pos_emb, type_emb, gamma, beta)


# speedup vs baseline: 1.1751x; 1.1707x over previous
import functools

import jax
import jax.numpy as jnp
from jax import lax
from jax.experimental import pallas as pl
from jax.experimental.pallas import tpu as pltpu
from jax.experimental.pallas import tpu_sc as plsc

_D = 768
_L = 16
_NSL = _D // _L
_B = 64
_S = 512
_NC = 2
_NS = 16
_NW = _NC * _NS
_POS_W = _S // _NW
_TOK_W = _B * _POS_W
_CHUNK = _POS_W
_NCHUNK = _B
_NBUF = 4
_EPS = 1e-12


def _rsqrt16(v):
    iv = plsc.bitcast(v, jnp.int32)
    ih = jnp.int32(0x5F3759DF) - lax.shift_right_logical(iv, 1)
    y = plsc.bitcast(ih, jnp.float32)
    for _ in range(3):
        y = y * (1.5 - 0.5 * v * y * y)
    return y


def _body(word_hbm, idx_hbm, tt_hbm, pos_hbm, type_hbm, gam_hbm, bet_hbm,
          out_hbm,
          idx_v, tt_v, base2_v, type_v, gam_v, bet_v, stats_v,
          rows0, rows1, rows2, rows3,
          gsem0, gsem1, gsem2, gsem3, wsem0, wsem1, wsem2, wsem3):
    wid = lax.axis_index("s") * _NC + lax.axis_index("c")
    bufs = (rows0, rows1, rows2, rows3)
    gsems = (gsem0, gsem1, gsem2, gsem3)
    wsems = (wsem0, wsem1, wsem2, wsem3)

    pltpu.sync_copy(idx_hbm.at[pl.ds(wid * _NCHUNK, _NCHUNK), :], idx_v)
    pltpu.sync_copy(tt_hbm.at[pl.ds(wid * _TOK_W, _TOK_W)],
                    tt_v.at[pl.ds(0, _TOK_W)])
    pltpu.sync_copy(pos_hbm.at[pl.ds(wid * _POS_W, _POS_W), :],
                    base2_v.at[0])
    pltpu.sync_copy(pos_hbm.at[pl.ds(wid * _POS_W, _POS_W), :],
                    base2_v.at[1])
    pltpu.sync_copy(type_hbm, type_v)
    pltpu.sync_copy(gam_hbm, gam_v)
    pltpu.sync_copy(bet_hbm, bet_v)

    def _init_d(d, _):
        sl = pl.ds(d * _L, _L)
        t0 = type_v[0, sl]
        t1 = type_v[1, sl]

        def _init_j(j, __):
            base2_v[0, j, sl] = base2_v[0, j, sl] + t0
            base2_v[1, j, sl] = base2_v[1, j, sl] + t1
            return 0

        lax.fori_loop(0, _POS_W, _init_j, 0)
        return 0

    lax.fori_loop(0, _NSL, _init_d, 0)

    zeros = jnp.zeros((_L,), jnp.float32)
    lanes = lax.iota(jnp.int32, _L)

    def _out_at(c):
        return out_hbm.at[c, pl.ds(wid * _POS_W, _POS_W), :]

    def _compute_chunk(rows, c):
        def _p1(ip, _):
            i0 = ip * 2
            i1 = i0 + 1
            tti0 = tt_v[pl.ds(c * _CHUNK + i0, _L)][0]
            tti1 = tt_v[pl.ds(c * _CHUNK + i1, _L)][0]
            s1_0 = s2_0 = s1_1 = s2_1 = zeros

            def _lds(d):
                sl = pl.ds(d * _L, _L)
                return (rows[i0, sl], base2_v[tti0, i0, sl],
                        rows[i1, sl], base2_v[tti1, i1, sl])

            cur = _lds(0)
            nxt = _lds(1)
            for d in range(_NSL):
                if d + 2 < _NSL:
                    nxt2 = _lds(d + 2)
                r0, b0, r1, b1 = cur
                sl = pl.ds(d * _L, _L)
                x0 = r0 + b0
                x1 = r1 + b1
                rows[i0, sl] = x0
                rows[i1, sl] = x1
                s1_0 = s1_0 + x0
                s2_0 = s2_0 + x0 * x0
                s1_1 = s1_1 + x1
                s2_1 = s2_1 + x1 * x1
                if d + 2 < _NSL:
                    cur, nxt = nxt, nxt2
                elif d + 1 < _NSL:
                    cur = nxt
            stats_v[i0, :] = s1_0
            stats_v[i1, :] = s1_1
            stats_v[_CHUNK + i0, :] = s2_0
            stats_v[_CHUNK + i1, :] = s2_1
            return 0

        lax.fori_loop(0, _CHUNK // 2, _p1, 0)

        tot1 = zeros
        tot2 = zeros
        for l in range(_L):
            cl = jnp.full((_L,), l, jnp.int32)
            tot1 = tot1 + plsc.load_gather(stats_v, [lanes, cl])
            tot2 = tot2 + plsc.load_gather(stats_v, [lanes + _CHUNK, cl])
        mean_v = tot1 * (1.0 / _D)
        var_v = jnp.maximum(tot2 * (1.0 / _D) - mean_v * mean_v, 0.0) + _EPS
        rst_v = _rsqrt16(var_v)
        rst_s = [rst_v[j] for j in range(_L)]
        mean_s = [mean_v[j] for j in range(_L)]

        def _p2(d, _):
            sl = pl.ds(d * _L, _L)
            gsl = gam_v[sl]
            bsl = bet_v[sl]
            x = rows[0, sl]
            for j in range(_L):
                if j + 1 < _L:
                    xn = rows[j + 1, sl]
                a = rst_s[j] * gsl
                cc2 = bsl - mean_s[j] * a
                rows[j, sl] = x * a + cc2
                if j + 1 < _L:
                    x = xn
            return 0

        lax.fori_loop(0, _NSL, _p2, 0, unroll=2)

    pltpu.async_copy(word_hbm.at[idx_v.at[0]], bufs[0], gsems[0])
    pltpu.async_copy(word_hbm.at[idx_v.at[1]], bufs[1], gsems[1])

    def _iter(cc, _):
        for k in range(_NBUF):
            c = cc * _NBUF + k
            k2 = (k + 2) % _NBUF
            pltpu.make_async_copy(word_hbm.at[idx_v.at[c]],
                                  bufs[k], gsems[k]).wait()
            _compute_chunk(bufs[k], c)
            pltpu.async_copy(bufs[k], _out_at(c), wsems[k])

            def _drain_w(c=c, k2=k2):
                pltpu.make_async_copy(bufs[k2], _out_at(c - 2),
                                      wsems[k2]).wait()

            def _start_g(c=c, k2=k2):
                pltpu.async_copy(word_hbm.at[idx_v.at[c + 2]],
                                 bufs[k2], gsems[k2])

            if k < 2:
                pl.when(cc > 0)(_drain_w)
                _start_g()
            else:
                _drain_w()
                pl.when(cc < _NCHUNK // _NBUF - 1)(_start_g)
        return 0

    lax.fori_loop(0, _NCHUNK // _NBUF, _iter, 0)

    pltpu.make_async_copy(bufs[2], _out_at(_NCHUNK - 2), wsems[2]).wait()
    pltpu.make_async_copy(bufs[3], _out_at(_NCHUNK - 1), wsems[3]).wait()


_emb_ln = functools.partial(
    pl.kernel,
    out_type=jax.ShapeDtypeStruct((_B, _S, _D), jnp.float32),
    mesh=plsc.VectorSubcoreMesh(core_axis_name="c", subcore_axis_name="s"),
    compiler_params=pltpu.CompilerParams(needs_layout_passes=False),
    scratch_types=[
        pltpu.VMEM((_NCHUNK, _CHUNK), jnp.int32),
        pltpu.VMEM((_TOK_W + _L,), jnp.int32),
        pltpu.VMEM((2, _POS_W, _D), jnp.float32),
        pltpu.VMEM((2, _D), jnp.float32),
        pltpu.VMEM((_D,), jnp.float32),
        pltpu.VMEM((_D,), jnp.float32),
        pltpu.VMEM((2 * _CHUNK, _L), jnp.float32),
        pltpu.VMEM((_CHUNK, _D), jnp.float32),
        pltpu.VMEM((_CHUNK, _D), jnp.float32),
        pltpu.VMEM((_CHUNK, _D), jnp.float32),
        pltpu.VMEM((_CHUNK, _D), jnp.float32),
        pltpu.SemaphoreType.DMA,
        pltpu.SemaphoreType.DMA,
        pltpu.SemaphoreType.DMA,
        pltpu.SemaphoreType.DMA,
        pltpu.SemaphoreType.DMA,
        pltpu.SemaphoreType.DMA,
        pltpu.SemaphoreType.DMA,
        pltpu.SemaphoreType.DMA,
    ],
)(_body)


def kernel(input_ids, token_type_ids, word_emb, pos_emb, type_emb, gamma, beta):
    ids_p = (input_ids.astype(jnp.int32)
             .reshape(_B, _NW, _POS_W)
             .transpose(1, 0, 2)
             .reshape(_NW * _NCHUNK, _CHUNK))
    tt_p = (token_type_ids.astype(jnp.int32)
            .reshape(_B, _NW, _POS_W)
            .transpose(1, 0, 2)
            .reshape(_NW * _TOK_W))
    return _emb_ln(word_emb, ids_p, tt_p, pos_emb, type_emb, gamma, beta)


# speedup vs baseline: 1.3081x; 1.1131x over previous
import functools

import jax
import jax.numpy as jnp
from jax import lax
from jax.experimental import pallas as pl
from jax.experimental.pallas import tpu as pltpu
from jax.experimental.pallas import tpu_sc as plsc

_D = 768
_L = 16
_NSL = _D // _L
_B = 64
_S = 512
_NC = 2
_NS = 16
_NW = _NC * _NS
_POS_W = _S // _NW
_TOK_W = _B * _POS_W
_CHUNK = _POS_W
_NCHUNK = _B
_NBUF = 4
_EPS = 1e-12


def _rsqrt16(v):
    iv = plsc.bitcast(v, jnp.int32)
    ih = jnp.int32(0x5F3759DF) - lax.shift_right_logical(iv, 1)
    y = plsc.bitcast(ih, jnp.float32)
    for _ in range(3):
        y = y * (1.5 - 0.5 * v * y * y)
    return y


def _body(word_hbm, idx_hbm, tt_hbm, pos_hbm, type_hbm, gam_hbm, bet_hbm,
          out_hbm,
          idx_v, tt_v, base2_v, type_v, gam_v, bet_v, stats_v,
          rows0, rows1, rows2, rows3,
          gsem0, gsem1, gsem2, gsem3, wsem0, wsem1, wsem2, wsem3):
    wid = lax.axis_index("s") * _NC + lax.axis_index("c")
    bufs = (rows0, rows1, rows2, rows3)
    gsems = (gsem0, gsem1, gsem2, gsem3)
    wsems = (wsem0, wsem1, wsem2, wsem3)

    pltpu.sync_copy(idx_hbm.at[pl.ds(wid * _NCHUNK, _NCHUNK), :], idx_v)
    pltpu.sync_copy(tt_hbm.at[pl.ds(wid * _TOK_W, _TOK_W)],
                    tt_v.at[pl.ds(0, _TOK_W)])
    pltpu.sync_copy(pos_hbm.at[pl.ds(wid * _POS_W, _POS_W), :],
                    base2_v.at[0])
    pltpu.sync_copy(pos_hbm.at[pl.ds(wid * _POS_W, _POS_W), :],
                    base2_v.at[1])
    pltpu.sync_copy(type_hbm, type_v)
    pltpu.sync_copy(gam_hbm, gam_v)
    pltpu.sync_copy(bet_hbm, bet_v)

    def _init_d(d, _):
        sl = pl.ds(d * _L, _L)
        t0 = type_v[0, sl]
        t1 = type_v[1, sl]

        def _init_j(j, __):
            base2_v[0, j, sl] = base2_v[0, j, sl] + t0
            base2_v[1, j, sl] = base2_v[1, j, sl] + t1
            return 0

        lax.fori_loop(0, _POS_W, _init_j, 0)
        return 0

    lax.fori_loop(0, _NSL, _init_d, 0)

    zeros = jnp.zeros((_L,), jnp.float32)
    lanes = lax.iota(jnp.int32, _L)

    def _out_at(c):
        return out_hbm.at[c, pl.ds(wid * _POS_W, _POS_W), :]

    def _compute_chunk(rows, c):
        def _p1(ip, _):
            i0 = ip * 2
            i1 = i0 + 1
            tti0 = tt_v[pl.ds(c * _CHUNK + i0, _L)][0]
            tti1 = tt_v[pl.ds(c * _CHUNK + i1, _L)][0]
            s1_0 = s2_0 = s1_1 = s2_1 = zeros

            def _lds(d):
                sl = pl.ds(d * _L, _L)
                return (rows[i0, sl], base2_v[tti0, i0, sl],
                        rows[i1, sl], base2_v[tti1, i1, sl])

            cur = _lds(0)
            nxt = _lds(1)
            for d in range(_NSL):
                if d + 2 < _NSL:
                    nxt2 = _lds(d + 2)
                r0, b0, r1, b1 = cur
                sl = pl.ds(d * _L, _L)
                x0 = r0 + b0
                x1 = r1 + b1
                rows[i0, sl] = x0
                rows[i1, sl] = x1
                s1_0 = s1_0 + x0
                s2_0 = s2_0 + x0 * x0
                s1_1 = s1_1 + x1
                s2_1 = s2_1 + x1 * x1
                if d + 2 < _NSL:
                    cur, nxt = nxt, nxt2
                elif d + 1 < _NSL:
                    cur = nxt
            stats_v[i0, :] = s1_0
            stats_v[i1, :] = s1_1
            stats_v[_CHUNK + i0, :] = s2_0
            stats_v[_CHUNK + i1, :] = s2_1
            return 0

        lax.fori_loop(0, _CHUNK // 2, _p1, 0)

        tot1 = zeros
        tot2 = zeros
        for l in range(_L):
            cl = jnp.full((_L,), l, jnp.int32)
            tot1 = tot1 + plsc.load_gather(stats_v, [lanes, cl])
            tot2 = tot2 + plsc.load_gather(stats_v, [lanes + _CHUNK, cl])
        mean_v = tot1 * (1.0 / _D)
        var_v = jnp.maximum(tot2 * (1.0 / _D) - mean_v * mean_v, 0.0) + _EPS
        rst_v = _rsqrt16(var_v)
        rst_s = [rst_v[j] for j in range(_L)]
        mean_s = [mean_v[j] for j in range(_L)]

        def _p2(d, _):
            sl = pl.ds(d * _L, _L)
            gsl = gam_v[sl]
            bsl = bet_v[sl]
            x = rows[0, sl]
            xn = rows[1, sl]
            for j in range(_L):
                if j + 2 < _L:
                    xn2 = rows[j + 2, sl]
                a = rst_s[j] * gsl
                cc2 = bsl - mean_s[j] * a
                rows[j, sl] = x * a + cc2
                if j + 2 < _L:
                    x, xn = xn, xn2
                elif j + 1 < _L:
                    x = xn
            return 0

        lax.fori_loop(0, _NSL, _p2, 0, unroll=2)

    pltpu.async_copy(word_hbm.at[idx_v.at[0]], bufs[0], gsems[0])
    pltpu.async_copy(word_hbm.at[idx_v.at[1]], bufs[1], gsems[1])

    def _iter(cc, _):
        for k in range(_NBUF):
            c = cc * _NBUF + k
            k2 = (k + 2) % _NBUF
            pltpu.make_async_copy(word_hbm.at[idx_v.at[c]],
                                  bufs[k], gsems[k]).wait()
            _compute_chunk(bufs[k], c)
            pltpu.async_copy(bufs[k], _out_at(c), wsems[k])

            def _drain_w(c=c, k2=k2):
                pltpu.make_async_copy(bufs[k2], _out_at(c - 2),
                                      wsems[k2]).wait()

            def _start_g(c=c, k2=k2):
                pltpu.async_copy(word_hbm.at[idx_v.at[c + 2]],
                                 bufs[k2], gsems[k2])

            if k < 2:
                pl.when(cc > 0)(_drain_w)
                _start_g()
            else:
                _drain_w()
                pl.when(cc < _NCHUNK // _NBUF - 1)(_start_g)
        return 0

    lax.fori_loop(0, _NCHUNK // _NBUF, _iter, 0)

    pltpu.make_async_copy(bufs[2], _out_at(_NCHUNK - 2), wsems[2]).wait()
    pltpu.make_async_copy(bufs[3], _out_at(_NCHUNK - 1), wsems[3]).wait()


_emb_ln = functools.partial(
    pl.kernel,
    out_type=jax.ShapeDtypeStruct((_B, _S, _D), jnp.float32),
    mesh=plsc.VectorSubcoreMesh(core_axis_name="c", subcore_axis_name="s"),
    compiler_params=pltpu.CompilerParams(needs_layout_passes=False),
    scratch_types=[
        pltpu.VMEM((_NCHUNK, _CHUNK), jnp.int32),
        pltpu.VMEM((_TOK_W + _L,), jnp.int32),
        pltpu.VMEM((2, _POS_W, _D), jnp.float32),
        pltpu.VMEM((2, _D), jnp.float32),
        pltpu.VMEM((_D,), jnp.float32),
        pltpu.VMEM((_D,), jnp.float32),
        pltpu.VMEM((2 * _CHUNK, _L), jnp.float32),
        pltpu.VMEM((_CHUNK, _D), jnp.float32),
        pltpu.VMEM((_CHUNK, _D), jnp.float32),
        pltpu.VMEM((_CHUNK, _D), jnp.float32),
        pltpu.VMEM((_CHUNK, _D), jnp.float32),
        pltpu.SemaphoreType.DMA,
        pltpu.SemaphoreType.DMA,
        pltpu.SemaphoreType.DMA,
        pltpu.SemaphoreType.DMA,
        pltpu.SemaphoreType.DMA,
        pltpu.SemaphoreType.DMA,
        pltpu.SemaphoreType.DMA,
        pltpu.SemaphoreType.DMA,
    ],
)(_body)


def kernel(input_ids, token_type_ids, word_emb, pos_emb, type_emb, gamma, beta):
    ids_p = (input_ids.astype(jnp.int32)
             .reshape(_B, _NW, _POS_W)
             .transpose(1, 0, 2)
             .reshape(_NW * _NCHUNK, _CHUNK))
    tt_p = (token_type_ids.astype(jnp.int32)
            .reshape(_B, _NW, _POS_W)
            .transpose(1, 0, 2)
            .reshape(_NW * _TOK_W))
    return _emb_ln(word_emb, ids_p, tt_p, pos_emb, type_emb, gamma, beta)


# speedup vs baseline: 1.4083x; 1.0766x over previous
import functools

import jax
import jax.numpy as jnp
from jax import lax
from jax.experimental import pallas as pl
from jax.experimental.pallas import tpu as pltpu
from jax.experimental.pallas import tpu_sc as plsc

_D = 768
_L = 16
_NSL = _D // _L
_B = 64
_S = 512
_NC = 2
_NS = 16
_NW = _NC * _NS
_POS_W = _S // _NW
_TOK_W = _B * _POS_W
_CHUNK = _POS_W
_NCHUNK = _B
_NBUF = 4
_EPS = 1e-12


def _rsqrt16(v):
    iv = plsc.bitcast(v, jnp.int32)
    ih = jnp.int32(0x5F3759DF) - lax.shift_right_logical(iv, 1)
    y = plsc.bitcast(ih, jnp.float32)
    for _ in range(3):
        y = y * (1.5 - 0.5 * v * y * y)
    return y


def _body(word_hbm, idx_hbm, tt_hbm, pos_hbm, type_hbm, gam_hbm, bet_hbm,
          out_hbm,
          idx_v, tt_v, base2_v, type_v, gam_v, bet_v, stats_v,
          rows0, rows1, rows2, rows3,
          gsem0, gsem1, gsem2, gsem3, wsem0, wsem1, wsem2, wsem3):
    wid = lax.axis_index("s") * _NC + lax.axis_index("c")
    bufs = (rows0, rows1, rows2, rows3)
    gsems = (gsem0, gsem1, gsem2, gsem3)
    wsems = (wsem0, wsem1, wsem2, wsem3)

    pltpu.sync_copy(idx_hbm.at[pl.ds(wid * _NCHUNK, _NCHUNK), :], idx_v)
    pltpu.sync_copy(tt_hbm.at[pl.ds(wid * _TOK_W, _TOK_W)],
                    tt_v.at[pl.ds(0, _TOK_W)])
    pltpu.sync_copy(pos_hbm.at[pl.ds(wid * _POS_W, _POS_W), :],
                    base2_v.at[0])
    pltpu.sync_copy(pos_hbm.at[pl.ds(wid * _POS_W, _POS_W), :],
                    base2_v.at[1])
    pltpu.sync_copy(type_hbm, type_v)
    pltpu.sync_copy(gam_hbm, gam_v)
    pltpu.sync_copy(bet_hbm, bet_v)

    def _init_d(d, _):
        sl = pl.ds(d * _L, _L)
        t0 = type_v[0, sl]
        t1 = type_v[1, sl]

        def _init_j(j, __):
            base2_v[0, j, sl] = base2_v[0, j, sl] + t0
            base2_v[1, j, sl] = base2_v[1, j, sl] + t1
            return 0

        lax.fori_loop(0, _POS_W, _init_j, 0)
        return 0

    lax.fori_loop(0, _NSL, _init_d, 0)

    zeros = jnp.zeros((_L,), jnp.float32)
    lanes = lax.iota(jnp.int32, _L)

    def _out_at(c):
        return out_hbm.at[c, pl.ds(wid * _POS_W, _POS_W), :]

    def _compute_chunk(rows, c):
        def _p1(ip, _):
            i0 = ip * 2
            i1 = i0 + 1
            tti0 = tt_v[pl.ds(c * _CHUNK + i0, _L)][0]
            tti1 = tt_v[pl.ds(c * _CHUNK + i1, _L)][0]
            s1_0 = s2_0 = s1_1 = s2_1 = zeros

            def _lds(d):
                sl = pl.ds(d * _L, _L)
                return (rows[i0, sl], base2_v[tti0, i0, sl],
                        rows[i1, sl], base2_v[tti1, i1, sl])

            cur = _lds(0)
            nxt = _lds(1)
            for d in range(_NSL):
                if d + 2 < _NSL:
                    nxt2 = _lds(d + 2)
                r0, b0, r1, b1 = cur
                sl = pl.ds(d * _L, _L)
                x0 = r0 + b0
                x1 = r1 + b1
                rows[i0, sl] = x0
                rows[i1, sl] = x1
                s1_0 = s1_0 + x0
                s2_0 = s2_0 + x0 * x0
                s1_1 = s1_1 + x1
                s2_1 = s2_1 + x1 * x1
                if d + 2 < _NSL:
                    cur, nxt = nxt, nxt2
                elif d + 1 < _NSL:
                    cur = nxt
            stats_v[i0, :] = s1_0
            stats_v[i1, :] = s1_1
            stats_v[_CHUNK + i0, :] = s2_0
            stats_v[_CHUNK + i1, :] = s2_1
            return 0

        lax.fori_loop(0, _CHUNK // 2, _p1, 0)

        tot1 = zeros
        tot2 = zeros
        for l in range(_L):
            cl = jnp.full((_L,), l, jnp.int32)
            tot1 = tot1 + plsc.load_gather(stats_v, [lanes, cl])
            tot2 = tot2 + plsc.load_gather(stats_v, [lanes + _CHUNK, cl])
        mean_v = tot1 * (1.0 / _D)
        var_v = jnp.maximum(tot2 * (1.0 / _D) - mean_v * mean_v, 0.0) + _EPS
        rst_v = _rsqrt16(var_v)
        rst_s = [rst_v[j] for j in range(_L)]
        mean_s = [mean_v[j] for j in range(_L)]

        def _p2(d, _):
            sl = pl.ds(d * _L, _L)
            gsl = gam_v[sl]
            bsl = bet_v[sl]
            pipe = [rows[0, sl], rows[1, sl], rows[2, sl]]
            for j in range(_L):
                if j + 3 < _L:
                    xn3 = rows[j + 3, sl]
                a = rst_s[j] * gsl
                cc2 = bsl - mean_s[j] * a
                rows[j, sl] = pipe[0] * a + cc2
                if j + 3 < _L:
                    pipe = [pipe[1], pipe[2], xn3]
                elif j + 1 < _L:
                    pipe = pipe[1:]
            return 0

        lax.fori_loop(0, _NSL, _p2, 0, unroll=2)

    pltpu.async_copy(word_hbm.at[idx_v.at[0]], bufs[0], gsems[0])
    pltpu.async_copy(word_hbm.at[idx_v.at[1]], bufs[1], gsems[1])

    def _iter(cc, _):
        for k in range(_NBUF):
            c = cc * _NBUF + k
            k2 = (k + 2) % _NBUF
            pltpu.make_async_copy(word_hbm.at[idx_v.at[c]],
                                  bufs[k], gsems[k]).wait()
            _compute_chunk(bufs[k], c)
            pltpu.async_copy(bufs[k], _out_at(c), wsems[k])

            def _drain_w(c=c, k2=k2):
                pltpu.make_async_copy(bufs[k2], _out_at(c - 2),
                                      wsems[k2]).wait()

            def _start_g(c=c, k2=k2):
                pltpu.async_copy(word_hbm.at[idx_v.at[c + 2]],
                                 bufs[k2], gsems[k2])

            if k < 2:
                pl.when(cc > 0)(_drain_w)
                _start_g()
            else:
                _drain_w()
                pl.when(cc < _NCHUNK // _NBUF - 1)(_start_g)
        return 0

    lax.fori_loop(0, _NCHUNK // _NBUF, _iter, 0)

    pltpu.make_async_copy(bufs[2], _out_at(_NCHUNK - 2), wsems[2]).wait()
    pltpu.make_async_copy(bufs[3], _out_at(_NCHUNK - 1), wsems[3]).wait()


_emb_ln = functools.partial(
    pl.kernel,
    out_type=jax.ShapeDtypeStruct((_B, _S, _D), jnp.float32),
    mesh=plsc.VectorSubcoreMesh(core_axis_name="c", subcore_axis_name="s"),
    compiler_params=pltpu.CompilerParams(needs_layout_passes=False),
    scratch_types=[
        pltpu.VMEM((_NCHUNK, _CHUNK), jnp.int32),
        pltpu.VMEM((_TOK_W + _L,), jnp.int32),
        pltpu.VMEM((2, _POS_W, _D), jnp.float32),
        pltpu.VMEM((2, _D), jnp.float32),
        pltpu.VMEM((_D,), jnp.float32),
        pltpu.VMEM((_D,), jnp.float32),
        pltpu.VMEM((2 * _CHUNK, _L), jnp.float32),
        pltpu.VMEM((_CHUNK, _D), jnp.float32),
        pltpu.VMEM((_CHUNK, _D), jnp.float32),
        pltpu.VMEM((_CHUNK, _D), jnp.float32),
        pltpu.VMEM((_CHUNK, _D), jnp.float32),
        pltpu.SemaphoreType.DMA,
        pltpu.SemaphoreType.DMA,
        pltpu.SemaphoreType.DMA,
        pltpu.SemaphoreType.DMA,
        pltpu.SemaphoreType.DMA,
        pltpu.SemaphoreType.DMA,
        pltpu.SemaphoreType.DMA,
        pltpu.SemaphoreType.DMA,
    ],
)(_body)


def kernel(input_ids, token_type_ids, word_emb, pos_emb, type_emb, gamma, beta):
    ids_p = (input_ids.astype(jnp.int32)
             .reshape(_B, _NW, _POS_W)
             .transpose(1, 0, 2)
             .reshape(_NW * _NCHUNK, _CHUNK))
    tt_p = (token_type_ids.astype(jnp.int32)
            .reshape(_B, _NW, _POS_W)
            .transpose(1, 0, 2)
            .reshape(_NW * _TOK_W))
    return _emb_ln(word_emb, ids_p, tt_p, pos_emb, type_emb, gamma, beta)


# speedup vs baseline: 1.4226x; 1.0102x over previous
import functools

import jax
import jax.numpy as jnp
from jax import lax
from jax.experimental import pallas as pl
from jax.experimental.pallas import tpu as pltpu
from jax.experimental.pallas import tpu_sc as plsc

_D = 768
_L = 16
_NSL = _D // _L
_B = 64
_S = 512
_NC = 2
_NS = 16
_NW = _NC * _NS
_POS_W = _S // _NW
_TOK_W = _B * _POS_W
_CHUNK = _POS_W
_NCHUNK = _B
_NBUF = 4
_EPS = 1e-12


def _rsqrt16(v):
    iv = plsc.bitcast(v, jnp.int32)
    ih = jnp.int32(0x5F3759DF) - lax.shift_right_logical(iv, 1)
    y = plsc.bitcast(ih, jnp.float32)
    for _ in range(3):
        y = y * (1.5 - 0.5 * v * y * y)
    return y


def _body(word_hbm, idx_hbm, tt_hbm, pos_hbm, type_hbm, gam_hbm, bet_hbm,
          out_hbm,
          idx_v, tt_v, base2_v, type_v, gam_v, bet_v, stats_v,
          rows0, rows1, rows2, rows3,
          gsem0, gsem1, gsem2, gsem3, wsem0, wsem1, wsem2, wsem3):
    wid = lax.axis_index("s") * _NC + lax.axis_index("c")
    bufs = (rows0, rows1, rows2, rows3)
    gsems = (gsem0, gsem1, gsem2, gsem3)
    wsems = (wsem0, wsem1, wsem2, wsem3)

    pltpu.sync_copy(idx_hbm.at[pl.ds(wid * _NCHUNK, _NCHUNK), :], idx_v)
    pltpu.sync_copy(tt_hbm.at[pl.ds(wid * _TOK_W, _TOK_W)],
                    tt_v.at[pl.ds(0, _TOK_W)])
    pltpu.sync_copy(pos_hbm.at[pl.ds(wid * _POS_W, _POS_W), :],
                    base2_v.at[0])
    pltpu.sync_copy(pos_hbm.at[pl.ds(wid * _POS_W, _POS_W), :],
                    base2_v.at[1])
    pltpu.sync_copy(type_hbm, type_v)
    pltpu.sync_copy(gam_hbm, gam_v)
    pltpu.sync_copy(bet_hbm, bet_v)

    def _init_d(d, _):
        sl = pl.ds(d * _L, _L)
        t0 = type_v[0, sl]
        t1 = type_v[1, sl]

        def _init_j(j, __):
            base2_v[0, j, sl] = base2_v[0, j, sl] + t0
            base2_v[1, j, sl] = base2_v[1, j, sl] + t1
            return 0

        lax.fori_loop(0, _POS_W, _init_j, 0)
        return 0

    lax.fori_loop(0, _NSL, _init_d, 0)

    zeros = jnp.zeros((_L,), jnp.float32)
    lanes = lax.iota(jnp.int32, _L)

    def _out_at(c):
        return out_hbm.at[c, pl.ds(wid * _POS_W, _POS_W), :]

    def _compute_chunk(rows, c):
        def _p1(ip, _):
            i0 = ip * 2
            i1 = i0 + 1
            tti0 = tt_v[pl.ds(c * _CHUNK + i0, _L)][0]
            tti1 = tt_v[pl.ds(c * _CHUNK + i1, _L)][0]
            s1_0 = s2_0 = s1_1 = s2_1 = zeros

            def _lds(d):
                sl = pl.ds(d * _L, _L)
                return (rows[i0, sl], base2_v[tti0, i0, sl],
                        rows[i1, sl], base2_v[tti1, i1, sl])

            pipe = [_lds(0), _lds(1), _lds(2)]
            for d in range(_NSL):
                if d + 3 < _NSL:
                    nxt3 = _lds(d + 3)
                r0, b0, r1, b1 = pipe[0]
                sl = pl.ds(d * _L, _L)
                x0 = r0 + b0
                x1 = r1 + b1
                rows[i0, sl] = x0
                rows[i1, sl] = x1
                s1_0 = s1_0 + x0
                s2_0 = s2_0 + x0 * x0
                s1_1 = s1_1 + x1
                s2_1 = s2_1 + x1 * x1
                if d + 3 < _NSL:
                    pipe = [pipe[1], pipe[2], nxt3]
                elif d + 1 < _NSL:
                    pipe = pipe[1:]
            stats_v[i0, :] = s1_0
            stats_v[i1, :] = s1_1
            stats_v[_CHUNK + i0, :] = s2_0
            stats_v[_CHUNK + i1, :] = s2_1
            return 0

        lax.fori_loop(0, _CHUNK // 2, _p1, 0)

        tot1 = zeros
        tot2 = zeros
        for l in range(_L):
            cl = jnp.full((_L,), l, jnp.int32)
            tot1 = tot1 + plsc.load_gather(stats_v, [lanes, cl])
            tot2 = tot2 + plsc.load_gather(stats_v, [lanes + _CHUNK, cl])
        mean_v = tot1 * (1.0 / _D)
        var_v = jnp.maximum(tot2 * (1.0 / _D) - mean_v * mean_v, 0.0) + _EPS
        rst_v = _rsqrt16(var_v)
        rst_s = [rst_v[j] for j in range(_L)]
        mean_s = [mean_v[j] for j in range(_L)]

        def _p2(d, _):
            sl = pl.ds(d * _L, _L)
            gsl = gam_v[sl]
            bsl = bet_v[sl]
            pipe = [rows[0, sl], rows[1, sl], rows[2, sl]]
            for j in range(_L):
                if j + 3 < _L:
                    xn3 = rows[j + 3, sl]
                a = rst_s[j] * gsl
                cc2 = bsl - mean_s[j] * a
                rows[j, sl] = pipe[0] * a + cc2
                if j + 3 < _L:
                    pipe = [pipe[1], pipe[2], xn3]
                elif j + 1 < _L:
                    pipe = pipe[1:]
            return 0

        lax.fori_loop(0, _NSL, _p2, 0, unroll=2)

    pltpu.async_copy(word_hbm.at[idx_v.at[0]], bufs[0], gsems[0])
    pltpu.async_copy(word_hbm.at[idx_v.at[1]], bufs[1], gsems[1])

    def _iter(cc, _):
        for k in range(_NBUF):
            c = cc * _NBUF + k
            k2 = (k + 2) % _NBUF
            pltpu.make_async_copy(word_hbm.at[idx_v.at[c]],
                                  bufs[k], gsems[k]).wait()
            _compute_chunk(bufs[k], c)
            pltpu.async_copy(bufs[k], _out_at(c), wsems[k])

            def _drain_w(c=c, k2=k2):
                pltpu.make_async_copy(bufs[k2], _out_at(c - 2),
                                      wsems[k2]).wait()

            def _start_g(c=c, k2=k2):
                pltpu.async_copy(word_hbm.at[idx_v.at[c + 2]],
                                 bufs[k2], gsems[k2])

            if k < 2:
                pl.when(cc > 0)(_drain_w)
                _start_g()
            else:
                _drain_w()
                pl.when(cc < _NCHUNK // _NBUF - 1)(_start_g)
        return 0

    lax.fori_loop(0, _NCHUNK // _NBUF, _iter, 0)

    pltpu.make_async_copy(bufs[2], _out_at(_NCHUNK - 2), wsems[2]).wait()
    pltpu.make_async_copy(bufs[3], _out_at(_NCHUNK - 1), wsems[3]).wait()


_emb_ln = functools.partial(
    pl.kernel,
    out_type=jax.ShapeDtypeStruct((_B, _S, _D), jnp.float32),
    mesh=plsc.VectorSubcoreMesh(core_axis_name="c", subcore_axis_name="s"),
    compiler_params=pltpu.CompilerParams(needs_layout_passes=False),
    scratch_types=[
        pltpu.VMEM((_NCHUNK, _CHUNK), jnp.int32),
        pltpu.VMEM((_TOK_W + _L,), jnp.int32),
        pltpu.VMEM((2, _POS_W, _D), jnp.float32),
        pltpu.VMEM((2, _D), jnp.float32),
        pltpu.VMEM((_D,), jnp.float32),
        pltpu.VMEM((_D,), jnp.float32),
        pltpu.VMEM((2 * _CHUNK, _L), jnp.float32),
        pltpu.VMEM((_CHUNK, _D), jnp.float32),
        pltpu.VMEM((_CHUNK, _D), jnp.float32),
        pltpu.VMEM((_CHUNK, _D), jnp.float32),
        pltpu.VMEM((_CHUNK, _D), jnp.float32),
        pltpu.SemaphoreType.DMA,
        pltpu.SemaphoreType.DMA,
        pltpu.SemaphoreType.DMA,
        pltpu.SemaphoreType.DMA,
        pltpu.SemaphoreType.DMA,
        pltpu.SemaphoreType.DMA,
        pltpu.SemaphoreType.DMA,
        pltpu.SemaphoreType.DMA,
    ],
)(_body)


def kernel(input_ids, token_type_ids, word_emb, pos_emb, type_emb, gamma, beta):
    ids_p = (input_ids.astype(jnp.int32)
             .reshape(_B, _NW, _POS_W)
             .transpose(1, 0, 2)
             .reshape(_NW * _NCHUNK, _CHUNK))
    tt_p = (token_type_ids.astype(jnp.int32)
            .reshape(_B, _NW, _POS_W)
            .transpose(1, 0, 2)
            .reshape(_NW * _TOK_W))
    return _emb_ln(word_emb, ids_p, tt_p, pos_emb, type_emb, gamma, beta)
